# baseline (speedup 1.0000x reference)
import functools

import jax
import jax.numpy as jnp
import numpy as np
from jax import lax
from jax.experimental import pallas as pl
from jax.experimental.pallas import tpu as pltpu
from jax.experimental.pallas import tpu_sc as plsc

N = 8192
NVREG = N // 16
ROWS = 128
NW = 32
ROWS_PER_W = ROWS // NW
NDIG = 256
_V = N * (N * N - 1) / 12.0
_INV_DENOM = float(1.0 / (_V + 1e-8))
_C = (N - 1) / 2.0
_MIN32 = np.int32(-(2 ** 31))


def _mono(bits):
    return bits ^ ((bits >> 31) | _MIN32)


def _sc_body(yt, yp, out, rt0, rp0, ka0, va0, h0,
             rt1, rp1, ka1, va1, h1, outbuf):
    wid = lax.axis_index("c") * 16 + lax.axis_index("s")
    lane = lax.iota(jnp.int32, 16)
    ones = jnp.ones((16,), jnp.int32)
    zeros16 = jnp.zeros((16,), jnp.int32)

    def zero_hists():
        @plsc.parallel_loop(0, NDIG, unroll=4)
        def zbody(j):
            h0[pl.ds(j * 16, 16)] = zeros16
            h1[pl.ds(j * 16, 16)] = zeros16

    def scan_hists():
        def sbody(jv, carries):
            ca, cb = carries
            for u in range(2):
                j = jv * 2 + u
                c0 = h0[pl.ds(j * 16, 16)]
                c1 = h1[pl.ds(j * 16, 16)]
                i0 = plsc.cumsum(c0)
                i1 = plsc.cumsum(c1)
                h0[pl.ds(j * 16, 16)] = i0 - c0 + ca
                h1[pl.ds(j * 16, 16)] = i1 - c1 + cb
                ca = ca + i0[15]
                cb = cb + i1[15]
            return ca, cb
        lax.fori_loop(0, NDIG // 2, sbody, (jnp.int32(0), jnp.int32(0)))

    def radix_pass(shift, lk, lv, kouts, vouts, acc0=None):
        zero_hists()
        mask = NDIG - 1

        U = 4

        def hbody(iv, c):
            i0 = iv * U
            ks = [lk[x](i0 + u) for u in range(U) for x in range(2)]
            slots = [((((k >> shift) & mask) << 4) | lane) for k in ks]
            for u in range(U):
                plsc.addupdate_scatter(h0, [slots[2 * u]], ones)
                plsc.addupdate_scatter(h1, [slots[2 * u + 1]], ones)
            return c
        lax.fori_loop(0, NVREG // U, hbody, jnp.int32(0))
        scan_hists()

        hists = (h0, h1)

        def pbody(iv, accs):
            accs = list(accs)
            i0 = iv * U
            ks = [[lk[x](i0 + u) for x in range(2)] for u in range(U)]
            slots = [[((((ks[u][x] >> shift) & mask) << 4) | lane)
                      for x in range(2)] for u in range(U)]
            vs = [[lv[x](i0 + u) for x in range(2)] for u in range(U)]
            for u in range(U):
                poss = [plsc.load_gather(hists[x], [slots[u][x]])
                        for x in range(2)]
                for x in range(2):
                    plsc.store_scatter(hists[x], [slots[u][x]],
                                       poss[x] + ones)
                if acc0 is not None:
                    for x in range(2):
                        accs[x] = accs[x] + (
                            (poss[x].astype(jnp.float32) - _C)
                            * (vs[u][x].astype(jnp.float32) - _C))
                else:
                    addrs = [(((poss[x] & (NVREG - 1)) << 4) | (poss[x] >> 9))
                             for x in range(2)]
                    for x in range(2):
                        if kouts[x] is not None:
                            plsc.store_scatter(kouts[x], [addrs[x]],
                                               ks[u][x])
                        plsc.store_scatter(vouts[x], [addrs[x]], vs[u][x])
            return tuple(accs)
        z = jnp.float32(0.0)
        init = (z, z) if acc0 is None else acc0
        return lax.fori_loop(0, NVREG // 2, pbody, init)

    def key_of(ref):
        return lambda i: ref[pl.ds(i * 16, 16)]

    def mono_of(ref):
        return lambda i: _mono(ref[pl.ds(i * 16, 16)])

    def pair_body(p, outacc):
        row = wid * ROWS_PER_W + p * 2
        pltpu.sync_copy(yt.at[row], rt0)
        pltpu.sync_copy(yp.at[row], rp0)
        pltpu.sync_copy(yt.at[row + 1], rt1)
        pltpu.sync_copy(yp.at[row + 1], rp1)

        radix_pass(0, (mono_of(rt0), mono_of(rt1)), (key_of(rp0), key_of(rp1)),
                   (ka0, ka1), (va0, va1))
        radix_pass(8, (key_of(ka0), key_of(ka1)), (key_of(va0), key_of(va1)),
                   (rt0, rt1), (rp0, rp1))
        radix_pass(16, (key_of(rt0), key_of(rt1)), (key_of(rp0), key_of(rp1)),
                   (ka0, ka1), (va0, va1))
        radix_pass(24, (key_of(ka0), key_of(ka1)), (key_of(va0), key_of(va1)),
                   (None, None), (rp0, rp1))

        genv = lambda i: (lane << 9) | i
        radix_pass(0, (mono_of(rp0), mono_of(rp1)), (genv, genv),
                   (ka0, ka1), (va0, va1))
        radix_pass(8, (key_of(ka0), key_of(ka1)), (key_of(va0), key_of(va1)),
                   (rt0, rt1), (rp0, rp1))
        radix_pass(16, (key_of(rt0), key_of(rt1)), (key_of(rp0), key_of(rp1)),
                   (ka0, ka1), (va0, va1))
        zf = jnp.zeros((16,), jnp.float32)
        acc = radix_pass(24, (key_of(ka0), key_of(ka1)),
                         (key_of(va0), key_of(va1)),
                         (None, None), (None, None), acc0=(zf, zf))

        c0 = jnp.sum(acc[0]) * jnp.float32(_INV_DENOM)
        c1 = jnp.sum(acc[1]) * jnp.float32(_INV_DENOM)
        outacc = outacc + jnp.where(lane == p * 2, c0, jnp.float32(0.0))
        return outacc + jnp.where(lane == p * 2 + 1, c1, jnp.float32(0.0))

    outacc = lax.fori_loop(0, ROWS_PER_W // 2, pair_body,
                           jnp.zeros((16,), jnp.float32))
    outbuf[...] = outacc
    pltpu.sync_copy(outbuf, out.at[wid])


_sc_spearman = functools.partial(
    pl.kernel,
    out_type=jax.ShapeDtypeStruct((NW, 16), jnp.float32),
    mesh=plsc.VectorSubcoreMesh(core_axis_name="c", subcore_axis_name="s"),
    compiler_params=pltpu.CompilerParams(needs_layout_passes=False),
    scratch_types=[
        pltpu.VMEM((N,), jnp.int32),
        pltpu.VMEM((N,), jnp.int32),
        pltpu.VMEM((N,), jnp.int32),
        pltpu.VMEM((N,), jnp.int32),
        pltpu.VMEM((NDIG * 16,), jnp.int32),
        pltpu.VMEM((N,), jnp.int32),
        pltpu.VMEM((N,), jnp.int32),
        pltpu.VMEM((N,), jnp.int32),
        pltpu.VMEM((N,), jnp.int32),
        pltpu.VMEM((NDIG * 16,), jnp.int32),
        pltpu.VMEM((16,), jnp.float32),
    ],
)(_sc_body)


@jax.jit
def kernel(y_pred, y_true):
    yt = jnp.transpose(y_true, (0, 2, 1)).reshape(ROWS, N)
    yp = jnp.transpose(y_pred, (0, 2, 1)).reshape(ROWS, N)
    yt_bits = jax.lax.bitcast_convert_type(yt, jnp.int32)
    yp_bits = jax.lax.bitcast_convert_type(yp, jnp.int32)
    out = _sc_spearman(yt_bits, yp_bits)
    return jnp.sum(out) / jnp.float32(ROWS)


# speedup vs baseline: 1.6400x; 1.6400x over previous
import functools

import jax
import jax.numpy as jnp
import numpy as np
from jax import lax
from jax.experimental import pallas as pl
from jax.experimental.pallas import tpu as pltpu
from jax.experimental.pallas import tpu_sc as plsc

N = 8192
NVREG = N // 16
ROWS = 128
NW = 32
ROWS_PER_W = ROWS // NW
NDIG = 256
_V = N * (N * N - 1) / 12.0
_INV_DENOM = float(1.0 / (_V + 1e-8))
_C = (N - 1) / 2.0
_MIN32 = np.int32(-(2 ** 31))


def _mono(bits):
    return bits ^ ((bits >> 31) | _MIN32)


def _sc_body(yt, yp, out, rt0, rp0, ka0, va0, h0,
             rt1, rp1, ka1, va1, h1, outbuf):
    wid = lax.axis_index("c") * 16 + lax.axis_index("s")
    lane = lax.iota(jnp.int32, 16)
    ones = jnp.ones((16,), jnp.int32)
    zeros16 = jnp.zeros((16,), jnp.int32)

    def zero_hists():
        @plsc.parallel_loop(0, NDIG, unroll=4)
        def zbody(j):
            h0[pl.ds(j * 16, 16)] = zeros16
            h1[pl.ds(j * 16, 16)] = zeros16

    def scan_hists():
        def sbody(jv, carries):
            ca, cb = carries
            for u in range(2):
                j = jv * 2 + u
                c0 = h0[pl.ds(j * 16, 16)]
                c1 = h1[pl.ds(j * 16, 16)]
                i0 = plsc.cumsum(c0)
                i1 = plsc.cumsum(c1)
                h0[pl.ds(j * 16, 16)] = i0 - c0 + ca
                h1[pl.ds(j * 16, 16)] = i1 - c1 + cb
                ca = ca + i0[15]
                cb = cb + i1[15]
            return ca, cb
        lax.fori_loop(0, NDIG // 2, sbody, (jnp.int32(0), jnp.int32(0)))

    def radix_pass(shift, lk, lv, kouts, vouts, acc0=None):
        zero_hists()
        mask = NDIG - 1

        U = 4

        def hbody(iv, c):
            i0 = iv * U
            ks = [lk[x](i0 + u) for u in range(U) for x in range(2)]
            slots = [((((k >> shift) & mask) << 4) | lane) for k in ks]
            for u in range(U):
                plsc.addupdate_scatter(h0, [slots[2 * u]], ones)
                plsc.addupdate_scatter(h1, [slots[2 * u + 1]], ones)
            return c
        lax.fori_loop(0, NVREG // U, hbody, jnp.int32(0))
        scan_hists()

        hists = (h0, h1)

        def pbody(iv, accs):
            accs = list(accs)
            i0 = iv * U
            ks = [[lk[x](i0 + u) for x in range(2)] for u in range(U)]
            slots = [[((((ks[u][x] >> shift) & mask) << 4) | lane)
                      for x in range(2)] for u in range(U)]
            vs = [[lv[x](i0 + u) for x in range(2)] for u in range(U)]
            for u in range(U):
                poss = [plsc.load_gather(hists[x], [slots[u][x]])
                        for x in range(2)]
                for x in range(2):
                    plsc.store_scatter(hists[x], [slots[u][x]],
                                       poss[x] + ones)
                if acc0 is not None:
                    for x in range(2):
                        accs[x] = accs[x] + (
                            (poss[x].astype(jnp.float32) - _C)
                            * (vs[u][x].astype(jnp.float32) - _C))
                else:
                    addrs = [(((poss[x] & (NVREG - 1)) << 4) | (poss[x] >> 9))
                             for x in range(2)]
                    for x in range(2):
                        if kouts[x] is not None:
                            plsc.store_scatter(kouts[x], [addrs[x]],
                                               ks[u][x])
                        plsc.store_scatter(vouts[x], [addrs[x]], vs[u][x])
            return tuple(accs)
        z = jnp.float32(0.0)
        init = (z, z) if acc0 is None else acc0
        return lax.fori_loop(0, NVREG // U, pbody, init)

    def key_of(ref):
        return lambda i: ref[pl.ds(i * 16, 16)]

    def mono_of(ref):
        return lambda i: _mono(ref[pl.ds(i * 16, 16)])

    def pair_body(p, outacc):
        row = wid * ROWS_PER_W + p * 2
        pltpu.sync_copy(yt.at[row], rt0)
        pltpu.sync_copy(yp.at[row], rp0)
        pltpu.sync_copy(yt.at[row + 1], rt1)
        pltpu.sync_copy(yp.at[row + 1], rp1)

        radix_pass(0, (mono_of(rt0), mono_of(rt1)), (key_of(rp0), key_of(rp1)),
                   (ka0, ka1), (va0, va1))
        radix_pass(8, (key_of(ka0), key_of(ka1)), (key_of(va0), key_of(va1)),
                   (rt0, rt1), (rp0, rp1))
        radix_pass(16, (key_of(rt0), key_of(rt1)), (key_of(rp0), key_of(rp1)),
                   (ka0, ka1), (va0, va1))
        radix_pass(24, (key_of(ka0), key_of(ka1)), (key_of(va0), key_of(va1)),
                   (None, None), (rp0, rp1))

        genv = lambda i: (lane << 9) | i
        radix_pass(0, (mono_of(rp0), mono_of(rp1)), (genv, genv),
                   (ka0, ka1), (va0, va1))
        radix_pass(8, (key_of(ka0), key_of(ka1)), (key_of(va0), key_of(va1)),
                   (rt0, rt1), (rp0, rp1))
        radix_pass(16, (key_of(rt0), key_of(rt1)), (key_of(rp0), key_of(rp1)),
                   (ka0, ka1), (va0, va1))
        zf = jnp.zeros((16,), jnp.float32)
        acc = radix_pass(24, (key_of(ka0), key_of(ka1)),
                         (key_of(va0), key_of(va1)),
                         (None, None), (None, None), acc0=(zf, zf))

        c0 = jnp.sum(acc[0]) * jnp.float32(_INV_DENOM)
        c1 = jnp.sum(acc[1]) * jnp.float32(_INV_DENOM)
        outacc = outacc + jnp.where(lane == p * 2, c0, jnp.float32(0.0))
        return outacc + jnp.where(lane == p * 2 + 1, c1, jnp.float32(0.0))

    outacc = lax.fori_loop(0, ROWS_PER_W // 2, pair_body,
                           jnp.zeros((16,), jnp.float32))
    outbuf[...] = outacc
    pltpu.sync_copy(outbuf, out.at[wid])


_sc_spearman = functools.partial(
    pl.kernel,
    out_type=jax.ShapeDtypeStruct((NW, 16), jnp.float32),
    mesh=plsc.VectorSubcoreMesh(core_axis_name="c", subcore_axis_name="s"),
    compiler_params=pltpu.CompilerParams(needs_layout_passes=False),
    scratch_types=[
        pltpu.VMEM((N,), jnp.int32),
        pltpu.VMEM((N,), jnp.int32),
        pltpu.VMEM((N,), jnp.int32),
        pltpu.VMEM((N,), jnp.int32),
        pltpu.VMEM((NDIG * 16,), jnp.int32),
        pltpu.VMEM((N,), jnp.int32),
        pltpu.VMEM((N,), jnp.int32),
        pltpu.VMEM((N,), jnp.int32),
        pltpu.VMEM((N,), jnp.int32),
        pltpu.VMEM((NDIG * 16,), jnp.int32),
        pltpu.VMEM((16,), jnp.float32),
    ],
)(_sc_body)


@jax.jit
def kernel(y_pred, y_true):
    yt = jnp.transpose(y_true, (0, 2, 1)).reshape(ROWS, N)
    yp = jnp.transpose(y_pred, (0, 2, 1)).reshape(ROWS, N)
    yt_bits = jax.lax.bitcast_convert_type(yt, jnp.int32)
    yp_bits = jax.lax.bitcast_convert_type(yp, jnp.int32)
    out = _sc_spearman(yt_bits, yp_bits)
    return jnp.sum(out) / jnp.float32(ROWS)


# speedup vs baseline: 1.7107x; 1.0431x over previous
import functools

import jax
import jax.numpy as jnp
import numpy as np
from jax import lax
from jax.experimental import pallas as pl
from jax.experimental.pallas import tpu as pltpu
from jax.experimental.pallas import tpu_sc as plsc

N = 8192
NVREG = N // 16
ROWS = 128
NW = 32
ROWS_PER_W = ROWS // NW
NDIG = 256
_V = N * (N * N - 1) / 12.0
_INV_DENOM = float(1.0 / (_V + 1e-8))
_C = (N - 1) / 2.0
_MIN32 = np.int32(-(2 ** 31))


def _mono(bits):
    return bits ^ ((bits >> 31) | _MIN32)


def _sc_body(yt, yp, out, rt0, rp0, ka0, va0, h0,
             rt1, rp1, ka1, va1, h1, outbuf):
    wid = lax.axis_index("c") * 16 + lax.axis_index("s")
    lane = lax.iota(jnp.int32, 16)
    ones = jnp.ones((16,), jnp.int32)
    zeros16 = jnp.zeros((16,), jnp.int32)

    def zero_hists():
        @plsc.parallel_loop(0, NDIG, unroll=4)
        def zbody(j):
            h0[pl.ds(j * 16, 16)] = zeros16
            h1[pl.ds(j * 16, 16)] = zeros16

    def scan_hists():
        def sbody(jv, carries):
            ca, cb = carries
            for u in range(2):
                j = jv * 2 + u
                c0 = h0[pl.ds(j * 16, 16)]
                c1 = h1[pl.ds(j * 16, 16)]
                i0 = plsc.cumsum(c0)
                i1 = plsc.cumsum(c1)
                h0[pl.ds(j * 16, 16)] = i0 - c0 + ca
                h1[pl.ds(j * 16, 16)] = i1 - c1 + cb
                ca = ca + i0[15]
                cb = cb + i1[15]
            return ca, cb
        lax.fori_loop(0, NDIG // 2, sbody, (jnp.int32(0), jnp.int32(0)))

    def radix_pass(shift, lk, lv, kouts, vouts, acc0=None):
        zero_hists()
        mask = NDIG - 1

        U = 8

        def hbody(iv, c):
            i0 = iv * U
            ks = [lk[x](i0 + u) for u in range(U) for x in range(2)]
            slots = [((((k >> shift) & mask) << 4) | lane) for k in ks]
            for u in range(U):
                plsc.addupdate_scatter(h0, [slots[2 * u]], ones)
                plsc.addupdate_scatter(h1, [slots[2 * u + 1]], ones)
            return c
        lax.fori_loop(0, NVREG // U, hbody, jnp.int32(0))
        scan_hists()

        hists = (h0, h1)

        def pbody(iv, accs):
            accs = list(accs)
            i0 = iv * U
            ks = [[lk[x](i0 + u) for x in range(2)] for u in range(U)]
            slots = [[((((ks[u][x] >> shift) & mask) << 4) | lane)
                      for x in range(2)] for u in range(U)]
            vs = [[lv[x](i0 + u) for x in range(2)] for u in range(U)]
            for u in range(U):
                poss = [plsc.load_gather(hists[x], [slots[u][x]])
                        for x in range(2)]
                for x in range(2):
                    plsc.store_scatter(hists[x], [slots[u][x]],
                                       poss[x] + ones)
                if acc0 is not None:
                    for x in range(2):
                        accs[x] = accs[x] + (
                            (poss[x].astype(jnp.float32) - _C)
                            * (vs[u][x].astype(jnp.float32) - _C))
                else:
                    addrs = [(((poss[x] & (NVREG - 1)) << 4) | (poss[x] >> 9))
                             for x in range(2)]
                    for x in range(2):
                        if kouts[x] is not None:
                            plsc.store_scatter(kouts[x], [addrs[x]],
                                               ks[u][x])
                        plsc.store_scatter(vouts[x], [addrs[x]], vs[u][x])
            return tuple(accs)
        z = jnp.float32(0.0)
        init = (z, z) if acc0 is None else acc0
        return lax.fori_loop(0, NVREG // U, pbody, init)

    def key_of(ref):
        return lambda i: ref[pl.ds(i * 16, 16)]

    def mono_of(ref):
        return lambda i: _mono(ref[pl.ds(i * 16, 16)])

    def pair_body(p, outacc):
        row = wid * ROWS_PER_W + p * 2
        pltpu.sync_copy(yt.at[row], rt0)
        pltpu.sync_copy(yp.at[row], rp0)
        pltpu.sync_copy(yt.at[row + 1], rt1)
        pltpu.sync_copy(yp.at[row + 1], rp1)

        radix_pass(0, (mono_of(rt0), mono_of(rt1)), (key_of(rp0), key_of(rp1)),
                   (ka0, ka1), (va0, va1))
        radix_pass(8, (key_of(ka0), key_of(ka1)), (key_of(va0), key_of(va1)),
                   (rt0, rt1), (rp0, rp1))
        radix_pass(16, (key_of(rt0), key_of(rt1)), (key_of(rp0), key_of(rp1)),
                   (ka0, ka1), (va0, va1))
        radix_pass(24, (key_of(ka0), key_of(ka1)), (key_of(va0), key_of(va1)),
                   (None, None), (rp0, rp1))

        genv = lambda i: (lane << 9) | i
        radix_pass(0, (mono_of(rp0), mono_of(rp1)), (genv, genv),
                   (ka0, ka1), (va0, va1))
        radix_pass(8, (key_of(ka0), key_of(ka1)), (key_of(va0), key_of(va1)),
                   (rt0, rt1), (rp0, rp1))
        radix_pass(16, (key_of(rt0), key_of(rt1)), (key_of(rp0), key_of(rp1)),
                   (ka0, ka1), (va0, va1))
        zf = jnp.zeros((16,), jnp.float32)
        acc = radix_pass(24, (key_of(ka0), key_of(ka1)),
                         (key_of(va0), key_of(va1)),
                         (None, None), (None, None), acc0=(zf, zf))

        c0 = jnp.sum(acc[0]) * jnp.float32(_INV_DENOM)
        c1 = jnp.sum(acc[1]) * jnp.float32(_INV_DENOM)
        outacc = outacc + jnp.where(lane == p * 2, c0, jnp.float32(0.0))
        return outacc + jnp.where(lane == p * 2 + 1, c1, jnp.float32(0.0))

    outacc = lax.fori_loop(0, ROWS_PER_W // 2, pair_body,
                           jnp.zeros((16,), jnp.float32))
    outbuf[...] = outacc
    pltpu.sync_copy(outbuf, out.at[wid])


_sc_spearman = functools.partial(
    pl.kernel,
    out_type=jax.ShapeDtypeStruct((NW, 16), jnp.float32),
    mesh=plsc.VectorSubcoreMesh(core_axis_name="c", subcore_axis_name="s"),
    compiler_params=pltpu.CompilerParams(needs_layout_passes=False),
    scratch_types=[
        pltpu.VMEM((N,), jnp.int32),
        pltpu.VMEM((N,), jnp.int32),
        pltpu.VMEM((N,), jnp.int32),
        pltpu.VMEM((N,), jnp.int32),
        pltpu.VMEM((NDIG * 16,), jnp.int32),
        pltpu.VMEM((N,), jnp.int32),
        pltpu.VMEM((N,), jnp.int32),
        pltpu.VMEM((N,), jnp.int32),
        pltpu.VMEM((N,), jnp.int32),
        pltpu.VMEM((NDIG * 16,), jnp.int32),
        pltpu.VMEM((16,), jnp.float32),
    ],
)(_sc_body)


@jax.jit
def kernel(y_pred, y_true):
    yt = jnp.transpose(y_true, (0, 2, 1)).reshape(ROWS, N)
    yp = jnp.transpose(y_pred, (0, 2, 1)).reshape(ROWS, N)
    yt_bits = jax.lax.bitcast_convert_type(yt, jnp.int32)
    yp_bits = jax.lax.bitcast_convert_type(yp, jnp.int32)
    out = _sc_spearman(yt_bits, yp_bits)
    return jnp.sum(out) / jnp.float32(ROWS)


# speedup vs baseline: 1.8306x; 1.0701x over previous
import functools

import jax
import jax.numpy as jnp
import numpy as np
from jax import lax
from jax.experimental import pallas as pl
from jax.experimental.pallas import tpu as pltpu
from jax.experimental.pallas import tpu_sc as plsc

N = 8192
NVREG = N // 16
ROWS = 128
NW = 32
ROWS_PER_W = ROWS // NW
NDIG = 256
_V = N * (N * N - 1) / 12.0
_INV_DENOM = float(1.0 / (_V + 1e-8))
_C = (N - 1) / 2.0
_MIN32 = np.int32(-(2 ** 31))


def _mono(bits):
    return bits ^ ((bits >> 31) | _MIN32)


def _sc_body(yt, yp, out, rt0, rp0, ka0, va0, h0,
             rt1, rp1, ka1, va1, h1, outbuf):
    wid = lax.axis_index("c") * 16 + lax.axis_index("s")
    lane = lax.iota(jnp.int32, 16)
    ones = jnp.ones((16,), jnp.int32)
    zeros16 = jnp.zeros((16,), jnp.int32)

    def zero_hists():
        @plsc.parallel_loop(0, NDIG, unroll=4)
        def zbody(j):
            h0[pl.ds(j * 16, 16)] = zeros16
            h1[pl.ds(j * 16, 16)] = zeros16

    def scan_hists():
        SU = 4

        def sbody(jv, carries):
            ca, cb = carries
            j0 = jv * SU
            c0s = [h0[pl.ds((j0 + u) * 16, 16)] for u in range(SU)]
            c1s = [h1[pl.ds((j0 + u) * 16, 16)] for u in range(SU)]
            i0s = [plsc.cumsum(c) for c in c0s]
            i1s = [plsc.cumsum(c) for c in c1s]
            for u in range(SU):
                h0[pl.ds((j0 + u) * 16, 16)] = i0s[u] - c0s[u] + ca
                h1[pl.ds((j0 + u) * 16, 16)] = i1s[u] - c1s[u] + cb
                ca = ca + i0s[u][15]
                cb = cb + i1s[u][15]
            return ca, cb
        lax.fori_loop(0, NDIG // SU, sbody, (jnp.int32(0), jnp.int32(0)))

    def radix_pass(shift, lk, lv, kouts, vouts, acc0=None):
        zero_hists()
        mask = NDIG - 1

        U = 8

        def hbody(iv, c):
            i0 = iv * U
            ks = [lk[x](i0 + u) for u in range(U) for x in range(2)]
            slots = [((((k >> shift) & mask) << 4) | lane) for k in ks]
            for u in range(U):
                plsc.addupdate_scatter(h0, [slots[2 * u]], ones)
                plsc.addupdate_scatter(h1, [slots[2 * u + 1]], ones)
            return c
        lax.fori_loop(0, NVREG // U, hbody, jnp.int32(0))
        scan_hists()

        hists = (h0, h1)

        def pbody(iv, accs):
            accs = list(accs)
            i0 = iv * U
            ks = [[lk[x](i0 + u) for x in range(2)] for u in range(U)]
            slots = [[((((ks[u][x] >> shift) & mask) << 4) | lane)
                      for x in range(2)] for u in range(U)]
            vs = [[lv[x](i0 + u) for x in range(2)] for u in range(U)]
            for u in range(U):
                poss = [plsc.load_gather(hists[x], [slots[u][x]])
                        for x in range(2)]
                for x in range(2):
                    plsc.store_scatter(hists[x], [slots[u][x]],
                                       poss[x] + ones)
                if acc0 is not None:
                    for x in range(2):
                        accs[x] = accs[x] + (
                            (poss[x].astype(jnp.float32) - _C)
                            * (vs[u][x].astype(jnp.float32) - _C))
                else:
                    addrs = [(((poss[x] & (NVREG - 1)) << 4) | (poss[x] >> 9))
                             for x in range(2)]
                    for x in range(2):
                        if kouts[x] is not None:
                            plsc.store_scatter(kouts[x], [addrs[x]],
                                               ks[u][x])
                        plsc.store_scatter(vouts[x], [addrs[x]], vs[u][x])
            return tuple(accs)
        z = jnp.float32(0.0)
        init = (z, z) if acc0 is None else acc0
        return lax.fori_loop(0, NVREG // U, pbody, init)

    def key_of(ref):
        return lambda i: ref[pl.ds(i * 16, 16)]

    def mono_of(ref):
        return lambda i: _mono(ref[pl.ds(i * 16, 16)])

    def pair_body(p, outacc):
        row = wid * ROWS_PER_W + p * 2
        pltpu.sync_copy(yt.at[row], rt0)
        pltpu.sync_copy(yp.at[row], rp0)
        pltpu.sync_copy(yt.at[row + 1], rt1)
        pltpu.sync_copy(yp.at[row + 1], rp1)

        radix_pass(0, (mono_of(rt0), mono_of(rt1)), (key_of(rp0), key_of(rp1)),
                   (ka0, ka1), (va0, va1))
        radix_pass(8, (key_of(ka0), key_of(ka1)), (key_of(va0), key_of(va1)),
                   (rt0, rt1), (rp0, rp1))
        radix_pass(16, (key_of(rt0), key_of(rt1)), (key_of(rp0), key_of(rp1)),
                   (ka0, ka1), (va0, va1))
        radix_pass(24, (key_of(ka0), key_of(ka1)), (key_of(va0), key_of(va1)),
                   (None, None), (rp0, rp1))

        genv = lambda i: (lane << 9) | i
        radix_pass(0, (mono_of(rp0), mono_of(rp1)), (genv, genv),
                   (ka0, ka1), (va0, va1))
        radix_pass(8, (key_of(ka0), key_of(ka1)), (key_of(va0), key_of(va1)),
                   (rt0, rt1), (rp0, rp1))
        radix_pass(16, (key_of(rt0), key_of(rt1)), (key_of(rp0), key_of(rp1)),
                   (ka0, ka1), (va0, va1))
        zf = jnp.zeros((16,), jnp.float32)
        acc = radix_pass(24, (key_of(ka0), key_of(ka1)),
                         (key_of(va0), key_of(va1)),
                         (None, None), (None, None), acc0=(zf, zf))

        c0 = jnp.sum(acc[0]) * jnp.float32(_INV_DENOM)
        c1 = jnp.sum(acc[1]) * jnp.float32(_INV_DENOM)
        outacc = outacc + jnp.where(lane == p * 2, c0, jnp.float32(0.0))
        return outacc + jnp.where(lane == p * 2 + 1, c1, jnp.float32(0.0))

    outacc = lax.fori_loop(0, ROWS_PER_W // 2, pair_body,
                           jnp.zeros((16,), jnp.float32))
    outbuf[...] = outacc
    pltpu.sync_copy(outbuf, out.at[wid])


_sc_spearman = functools.partial(
    pl.kernel,
    out_type=jax.ShapeDtypeStruct((NW, 16), jnp.float32),
    mesh=plsc.VectorSubcoreMesh(core_axis_name="c", subcore_axis_name="s"),
    compiler_params=pltpu.CompilerParams(needs_layout_passes=False),
    scratch_types=[
        pltpu.VMEM((N,), jnp.int32),
        pltpu.VMEM((N,), jnp.int32),
        pltpu.VMEM((N,), jnp.int32),
        pltpu.VMEM((N,), jnp.int32),
        pltpu.VMEM((NDIG * 16,), jnp.int32),
        pltpu.VMEM((N,), jnp.int32),
        pltpu.VMEM((N,), jnp.int32),
        pltpu.VMEM((N,), jnp.int32),
        pltpu.VMEM((N,), jnp.int32),
        pltpu.VMEM((NDIG * 16,), jnp.int32),
        pltpu.VMEM((16,), jnp.float32),
    ],
)(_sc_body)


@jax.jit
def kernel(y_pred, y_true):
    yt = jnp.transpose(y_true, (0, 2, 1)).reshape(ROWS, N)
    yp = jnp.transpose(y_pred, (0, 2, 1)).reshape(ROWS, N)
    yt_bits = jax.lax.bitcast_convert_type(yt, jnp.int32)
    yp_bits = jax.lax.bitcast_convert_type(yp, jnp.int32)
    out = _sc_spearman(yt_bits, yp_bits)
    return jnp.sum(out) / jnp.float32(ROWS)


# speedup vs baseline: 1.8333x; 1.0015x over previous
import functools

import jax
import jax.numpy as jnp
import numpy as np
from jax import lax
from jax.experimental import pallas as pl
from jax.experimental.pallas import tpu as pltpu
from jax.experimental.pallas import tpu_sc as plsc

N = 8192
NVREG = N // 16
ROWS = 128
NW = 32
ROWS_PER_W = ROWS // NW
NDIG = 256
_V = N * (N * N - 1) / 12.0
_INV_DENOM = float(1.0 / (_V + 1e-8))
_C = (N - 1) / 2.0
_MIN32 = np.int32(-(2 ** 31))


def _mono(bits):
    return bits ^ ((bits >> 31) | _MIN32)


def _sc_body(yt, yp, out, rt0, rp0, ka0, va0, h0,
             rt1, rp1, ka1, va1, h1, outbuf, dsem):
    wid = lax.axis_index("c") * 16 + lax.axis_index("s")
    lane = lax.iota(jnp.int32, 16)
    ones = jnp.ones((16,), jnp.int32)
    zeros16 = jnp.zeros((16,), jnp.int32)

    def zero_hists():
        @plsc.parallel_loop(0, NDIG, unroll=4)
        def zbody(j):
            h0[pl.ds(j * 16, 16)] = zeros16
            h1[pl.ds(j * 16, 16)] = zeros16

    def scan_hists():
        SU = 4

        def sbody(jv, carries):
            ca, cb = carries
            j0 = jv * SU
            c0s = [h0[pl.ds((j0 + u) * 16, 16)] for u in range(SU)]
            c1s = [h1[pl.ds((j0 + u) * 16, 16)] for u in range(SU)]
            i0s = [plsc.cumsum(c) for c in c0s]
            i1s = [plsc.cumsum(c) for c in c1s]
            for u in range(SU):
                h0[pl.ds((j0 + u) * 16, 16)] = i0s[u] - c0s[u] + ca
                h1[pl.ds((j0 + u) * 16, 16)] = i1s[u] - c1s[u] + cb
                ca = ca + i0s[u][15]
                cb = cb + i1s[u][15]
            return ca, cb
        lax.fori_loop(0, NDIG // SU, sbody, (jnp.int32(0), jnp.int32(0)))

    def radix_pass(shift, lk, lv, kouts, vouts, acc0=None):
        zero_hists()
        mask = NDIG - 1

        U = 8

        def hbody(iv, c):
            i0 = iv * U
            ks = [lk[x](i0 + u) for u in range(U) for x in range(2)]
            slots = [((((k >> shift) & mask) << 4) | lane) for k in ks]
            for u in range(U):
                plsc.addupdate_scatter(h0, [slots[2 * u]], ones)
                plsc.addupdate_scatter(h1, [slots[2 * u + 1]], ones)
            return c
        lax.fori_loop(0, NVREG // U, hbody, jnp.int32(0))
        scan_hists()

        hists = (h0, h1)

        def pbody(iv, accs):
            accs = list(accs)
            i0 = iv * U
            ks = [[lk[x](i0 + u) for x in range(2)] for u in range(U)]
            slots = [[((((ks[u][x] >> shift) & mask) << 4) | lane)
                      for x in range(2)] for u in range(U)]
            vs = [[lv[x](i0 + u) for x in range(2)] for u in range(U)]
            for u in range(U):
                poss = [plsc.load_gather(hists[x], [slots[u][x]])
                        for x in range(2)]
                for x in range(2):
                    plsc.store_scatter(hists[x], [slots[u][x]],
                                       poss[x] + ones)
                if acc0 is not None:
                    for x in range(2):
                        accs[x] = accs[x] + (
                            (poss[x].astype(jnp.float32) - _C)
                            * (vs[u][x].astype(jnp.float32) - _C))
                else:
                    addrs = [(((poss[x] & (NVREG - 1)) << 4) | (poss[x] >> 9))
                             for x in range(2)]
                    for x in range(2):
                        if kouts[x] is not None:
                            plsc.store_scatter(kouts[x], [addrs[x]],
                                               ks[u][x])
                        plsc.store_scatter(vouts[x], [addrs[x]], vs[u][x])
            return tuple(accs)
        z = jnp.float32(0.0)
        init = (z, z) if acc0 is None else acc0
        return lax.fori_loop(0, NVREG // U, pbody, init)

    def key_of(ref):
        return lambda i: ref[pl.ds(i * 16, 16)]

    def mono_of(ref):
        return lambda i: _mono(ref[pl.ds(i * 16, 16)])

    outacc = jnp.zeros((16,), jnp.float32)
    prefetch = []
    for p in range(ROWS_PER_W // 2):
        row = wid * ROWS_PER_W + p * 2
        if p == 0:
            pltpu.sync_copy(yt.at[row], rt0)
            pltpu.sync_copy(yp.at[row], rp0)
            pltpu.sync_copy(yt.at[row + 1], rt1)
            pltpu.sync_copy(yp.at[row + 1], rp1)
        else:
            for hcp in prefetch:
                hcp.wait()

        radix_pass(0, (mono_of(rt0), mono_of(rt1)), (key_of(rp0), key_of(rp1)),
                   (ka0, ka1), (va0, va1))
        radix_pass(8, (key_of(ka0), key_of(ka1)), (key_of(va0), key_of(va1)),
                   (rt0, rt1), (rp0, rp1))
        radix_pass(16, (key_of(rt0), key_of(rt1)), (key_of(rp0), key_of(rp1)),
                   (ka0, ka1), (va0, va1))
        radix_pass(24, (key_of(ka0), key_of(ka1)), (key_of(va0), key_of(va1)),
                   (None, None), (rp0, rp1))

        genv = lambda i: (lane << 9) | i
        radix_pass(0, (mono_of(rp0), mono_of(rp1)), (genv, genv),
                   (ka0, ka1), (va0, va1))
        radix_pass(8, (key_of(ka0), key_of(ka1)), (key_of(va0), key_of(va1)),
                   (rt0, rt1), (rp0, rp1))
        radix_pass(16, (key_of(rt0), key_of(rt1)), (key_of(rp0), key_of(rp1)),
                   (ka0, ka1), (va0, va1))
        if p == 0 and ROWS_PER_W // 2 > 1:
            prefetch = [
                pltpu.async_copy(yt.at[row + 2], rt0, dsem),
                pltpu.async_copy(yp.at[row + 2], rp0, dsem),
                pltpu.async_copy(yt.at[row + 3], rt1, dsem),
                pltpu.async_copy(yp.at[row + 3], rp1, dsem),
            ]
        zf = jnp.zeros((16,), jnp.float32)
        acc = radix_pass(24, (key_of(ka0), key_of(ka1)),
                         (key_of(va0), key_of(va1)),
                         (None, None), (None, None), acc0=(zf, zf))

        c0 = jnp.sum(acc[0]) * jnp.float32(_INV_DENOM)
        c1 = jnp.sum(acc[1]) * jnp.float32(_INV_DENOM)
        outacc = outacc + jnp.where(lane == p * 2, c0, jnp.float32(0.0))
        outacc = outacc + jnp.where(lane == p * 2 + 1, c1, jnp.float32(0.0))

    outbuf[...] = outacc
    pltpu.sync_copy(outbuf, out.at[wid])


_sc_spearman = functools.partial(
    pl.kernel,
    out_type=jax.ShapeDtypeStruct((NW, 16), jnp.float32),
    mesh=plsc.VectorSubcoreMesh(core_axis_name="c", subcore_axis_name="s"),
    compiler_params=pltpu.CompilerParams(needs_layout_passes=False),
    scratch_types=[
        pltpu.VMEM((N,), jnp.int32),
        pltpu.VMEM((N,), jnp.int32),
        pltpu.VMEM((N,), jnp.int32),
        pltpu.VMEM((N,), jnp.int32),
        pltpu.VMEM((NDIG * 16,), jnp.int32),
        pltpu.VMEM((N,), jnp.int32),
        pltpu.VMEM((N,), jnp.int32),
        pltpu.VMEM((N,), jnp.int32),
        pltpu.VMEM((N,), jnp.int32),
        pltpu.VMEM((NDIG * 16,), jnp.int32),
        pltpu.VMEM((16,), jnp.float32),
        pltpu.SemaphoreType.DMA,
    ],
)(_sc_body)


@jax.jit
def kernel(y_pred, y_true):
    yt = jnp.transpose(y_true, (0, 2, 1)).reshape(ROWS, N)
    yp = jnp.transpose(y_pred, (0, 2, 1)).reshape(ROWS, N)
    yt_bits = jax.lax.bitcast_convert_type(yt, jnp.int32)
    yp_bits = jax.lax.bitcast_convert_type(yp, jnp.int32)
    out = _sc_spearman(yt_bits, yp_bits)
    return jnp.sum(out) / jnp.float32(ROWS)
